# Initial kernel scaffold; baseline (speedup 1.0000x reference)
import functools

import jax
import jax.numpy as jnp
from jax import lax
from jax.experimental import pallas as pl
from jax.experimental.pallas import tpu as pltpu
from jax.experimental.pallas import tpu_sc as plsc

N_NODES = 10000
N_PAD = 10240
N_EDGES = 320000
D = 128
NUM_CLASSES = 64

NC = 2
NS = 16
NW = NC * NS
EPT = N_EDGES // NW
CHUNK = 125
NCH = EPT // CHUNK
ROWS_PT = N_PAD // NS

_f32 = jnp.float32


def _make_sc_agg(with_deg):
  mesh = plsc.VectorSubcoreMesh(core_axis_name="c", subcore_axis_name="s")

  out_type = [jax.ShapeDtypeStruct((NC, N_PAD, D), _f32)]
  scratch = [
      pltpu.VMEM((NCH, CHUNK), jnp.int32),
      pltpu.VMEM((NCH, CHUNK), jnp.int32),
      pltpu.VMEM((CHUNK, D), _f32),
      pltpu.VMEM((128, D), _f32),
      pltpu.VMEM_SHARED((N_PAD, D), _f32),
      pltpu.SemaphoreType.DMA,
  ]
  if with_deg:
    out_type.append(jax.ShapeDtypeStruct((NC, N_PAD), _f32))
    scratch += [
        pltpu.VMEM((128,), _f32),
        pltpu.VMEM((ROWS_PT,), _f32),
        pltpu.VMEM_SHARED((N_PAD,), _f32),
    ]

  def body(table_hbm, src_hbm, dst_hbm, agg_out, *rest):
    if with_deg:
      deg_out, src_t, dst_t, rows_v, zrow, agg_sh, sem, ones_v, zvec, deg_sh = rest
    else:
      src_t, dst_t, rows_v, zrow, agg_sh, sem = rest
    c = lax.axis_index("c")
    s = lax.axis_index("s")
    wid = s * NC + c

    def zfill_row(i, _):
      def zfill_col(j, _):
        zrow[i, pl.ds(j * 16, 16)] = jnp.zeros((16,), _f32)
        return 0
      return lax.fori_loop(0, D // 16, zfill_col, 0)
    lax.fori_loop(0, 128, zfill_row, 0)

    r0 = s * ROWS_PT
    for r in range(ROWS_PT // 128):
      pltpu.sync_copy(zrow, agg_sh.at[pl.ds(r0 + r * 128, 128), :])

    if with_deg:
      def ofill(j, _):
        ones_v[pl.ds(j * 16, 16)] = jnp.ones((16,), _f32)
        return 0
      lax.fori_loop(0, 128 // 16, ofill, 0)
      def zvfill(j, _):
        zvec[pl.ds(j * 16, 16)] = jnp.zeros((16,), _f32)
        return 0
      lax.fori_loop(0, ROWS_PT // 16, zvfill, 0)
      pltpu.sync_copy(zvec, deg_sh.at[pl.ds(r0, ROWS_PT)])

    pltpu.sync_copy(src_hbm.at[wid], src_t)
    pltpu.sync_copy(dst_hbm.at[wid], dst_t)

    plsc.subcore_barrier()

    def step(i, _):
      pltpu.async_copy(table_hbm.at[src_t.at[i]], rows_v, sem).wait()
      pltpu.sync_copy(rows_v, agg_sh.at[dst_t.at[i]], add=True)
      if with_deg:
        pltpu.sync_copy(ones_v.at[pl.ds(0, CHUNK)], deg_sh.at[dst_t.at[i]],
                        add=True)
      return 0
    lax.fori_loop(0, NCH, step, 0)

    plsc.subcore_barrier()

    pltpu.sync_copy(agg_sh.at[pl.ds(r0, ROWS_PT), :],
                    agg_out.at[c, pl.ds(r0, ROWS_PT), :])
    if with_deg:
      pltpu.sync_copy(deg_sh.at[pl.ds(r0, ROWS_PT)],
                      deg_out.at[c, pl.ds(r0, ROWS_PT)])

  return pl.kernel(body, out_type=out_type, mesh=mesh, scratch_types=scratch)


_sc_agg_deg = _make_sc_agg(True)
_sc_agg = _make_sc_agg(False)

ROWS_B = 256
GRID = N_PAD // ROWS_B


def _tc1_body(p_ref, d_ref, w_ref, b_ref, o_ref):
  a = p_ref[0] + p_ref[1]
  d = jnp.maximum(d_ref[0] + d_ref[1], 1.0)
  a = a / d
  h = jnp.dot(a, w_ref[...], preferred_element_type=_f32) + b_ref[...]
  o_ref[...] = jnp.maximum(h, 0.0)


_tc1 = pl.pallas_call(
    _tc1_body,
    grid=(GRID,),
    in_specs=[
        pl.BlockSpec((NC, ROWS_B, D), lambda j: (0, j, 0)),
        pl.BlockSpec((NC, ROWS_B, 1), lambda j: (0, j, 0)),
        pl.BlockSpec((D, D), lambda j: (0, 0)),
        pl.BlockSpec((1, D), lambda j: (0, 0)),
    ],
    out_specs=pl.BlockSpec((ROWS_B, D), lambda j: (j, 0)),
    out_shape=jax.ShapeDtypeStruct((N_PAD, D), _f32),
)


def _tc2_body(p_ref, d_ref, w2_ref, b2_ref, wd_ref, bd_ref, o_ref):
  a = p_ref[0] + p_ref[1]
  d = jnp.maximum(d_ref[0] + d_ref[1], 1.0)
  a = a / d
  h = jnp.maximum(
      jnp.dot(a, w2_ref[...], preferred_element_type=_f32) + b2_ref[...], 0.0)
  lg = jnp.dot(h, wd_ref[...], preferred_element_type=_f32) + bd_ref[...]
  m = jnp.max(lg, axis=-1, keepdims=True)
  e = jnp.exp(lg - m)
  o_ref[...] = e / jnp.sum(e, axis=-1, keepdims=True)


_tc2 = pl.pallas_call(
    _tc2_body,
    grid=(GRID,),
    in_specs=[
        pl.BlockSpec((NC, ROWS_B, D), lambda j: (0, j, 0)),
        pl.BlockSpec((NC, ROWS_B, 1), lambda j: (0, j, 0)),
        pl.BlockSpec((D, D), lambda j: (0, 0)),
        pl.BlockSpec((1, D), lambda j: (0, 0)),
        pl.BlockSpec((D, NUM_CLASSES), lambda j: (0, 0)),
        pl.BlockSpec((1, NUM_CLASSES), lambda j: (0, 0)),
    ],
    out_specs=pl.BlockSpec((ROWS_B, NUM_CLASSES), lambda j: (j, 0)),
    out_shape=jax.ShapeDtypeStruct((N_PAD, NUM_CLASSES), _f32),
)


def kernel(x, edge_index, W1, b1, W2, b2, Wd, bd):
  src = edge_index[0].reshape(NW, NCH, CHUNK)
  dst = edge_index[1].reshape(NW, NCH, CHUNK)
  agg1p, degp = _sc_agg_deg(x, src, dst)
  degp = degp.reshape(NC, N_PAD, 1)
  h1 = _tc1(agg1p, degp, W1, b1.reshape(1, D))
  agg2p = _sc_agg(h1, src, dst)
  out = _tc2(agg2p, degp, W2, b2.reshape(1, D), Wd, bd.reshape(1, NUM_CLASSES))
  return out[:N_NODES]


# speedup vs baseline: 8.1495x; 8.1495x over previous
import functools

import jax
import jax.numpy as jnp
from jax import lax
from jax.experimental import pallas as pl
from jax.experimental.pallas import tpu as pltpu
from jax.experimental.pallas import tpu_sc as plsc

N_NODES = 10000
N_PAD = 10240
N_EDGES = 320000
D = 128
NUM_CLASSES = 64

NC = 2
NS = 16
NW = NC * NS
EPT = N_EDGES // NW
CHUNK = 125
NCH = EPT // CHUNK
ROWS_PT = N_PAD // NS

_f32 = jnp.float32


def _make_sc_agg(with_deg):
  mesh = plsc.VectorSubcoreMesh(core_axis_name="c", subcore_axis_name="s")

  out_type = [jax.ShapeDtypeStruct((NC, N_PAD, D), _f32)]
  scratch = [
      pltpu.VMEM((NCH, CHUNK), jnp.int32),
      pltpu.VMEM((NCH, CHUNK), jnp.int32),
      pltpu.VMEM((CHUNK, D), _f32),
      pltpu.VMEM((16, D), _f32),
      pltpu.VMEM_SHARED((N_PAD, D), _f32),
      pltpu.SemaphoreType.DMA,
  ]
  if with_deg:
    out_type.append(jax.ShapeDtypeStruct((NC, N_PAD), _f32))
    scratch += [
        pltpu.VMEM((128,), _f32),
        pltpu.VMEM((ROWS_PT,), _f32),
        pltpu.VMEM_SHARED((N_PAD,), _f32),
    ]

  def body(table_hbm, src_hbm, dst_hbm, agg_out, *rest):
    if with_deg:
      deg_out, src_t, dst_t, rows_v, zrow, agg_sh, sem, ones_v, zvec, deg_sh = rest
    else:
      src_t, dst_t, rows_v, zrow, agg_sh, sem = rest
    c = lax.axis_index("c")
    s = lax.axis_index("s")
    wid = s * NC + c

    def zfill_row(i, _):
      def zfill_col(j, _):
        zrow[i, pl.ds(j * 16, 16)] = jnp.zeros((16,), _f32)
        return 0
      return lax.fori_loop(0, D // 16, zfill_col, 0)
    lax.fori_loop(0, 16, zfill_row, 0)

    r0 = s * ROWS_PT
    def zcopy(r, _):
      pltpu.sync_copy(zrow, agg_sh.at[pl.ds(r0 + r * 16, 16), :])
      return 0
    lax.fori_loop(0, ROWS_PT // 16, zcopy, 0)

    if with_deg:
      def ofill(j, _):
        ones_v[pl.ds(j * 16, 16)] = jnp.ones((16,), _f32)
        return 0
      lax.fori_loop(0, 128 // 16, ofill, 0)
      def zvfill(j, _):
        zvec[pl.ds(j * 16, 16)] = jnp.zeros((16,), _f32)
        return 0
      lax.fori_loop(0, ROWS_PT // 16, zvfill, 0)
      pltpu.sync_copy(zvec, deg_sh.at[pl.ds(r0, ROWS_PT)])

    pltpu.sync_copy(src_hbm.at[wid], src_t)
    pltpu.sync_copy(dst_hbm.at[wid], dst_t)

    plsc.subcore_barrier()

    def step(i, _):
      pltpu.async_copy(table_hbm.at[src_t.at[i]], rows_v, sem).wait()
      pltpu.sync_copy(rows_v, agg_sh.at[dst_t.at[i]], add=True)
      if with_deg:
        pltpu.sync_copy(ones_v.at[pl.ds(0, CHUNK)], deg_sh.at[dst_t.at[i]],
                        add=True)
      return 0
    lax.fori_loop(0, NCH, step, 0)

    plsc.subcore_barrier()

    pltpu.sync_copy(agg_sh.at[pl.ds(r0, ROWS_PT), :],
                    agg_out.at[c, pl.ds(r0, ROWS_PT), :])
    if with_deg:
      pltpu.sync_copy(deg_sh.at[pl.ds(r0, ROWS_PT)],
                      deg_out.at[c, pl.ds(r0, ROWS_PT)])

  return pl.kernel(body, out_type=out_type, mesh=mesh, scratch_types=scratch)


_sc_agg_deg = _make_sc_agg(True)
_sc_agg = _make_sc_agg(False)

ROWS_B = 256
GRID = N_PAD // ROWS_B


def _tc1_body(p_ref, d_ref, w_ref, b_ref, o_ref):
  a = p_ref[0] + p_ref[1]
  d = jnp.maximum(d_ref[0] + d_ref[1], 1.0)
  a = a / d
  h = jnp.dot(a, w_ref[...], preferred_element_type=_f32) + b_ref[...]
  o_ref[...] = jnp.maximum(h, 0.0)


_tc1 = pl.pallas_call(
    _tc1_body,
    grid=(GRID,),
    in_specs=[
        pl.BlockSpec((NC, ROWS_B, D), lambda j: (0, j, 0)),
        pl.BlockSpec((NC, ROWS_B, 1), lambda j: (0, j, 0)),
        pl.BlockSpec((D, D), lambda j: (0, 0)),
        pl.BlockSpec((1, D), lambda j: (0, 0)),
    ],
    out_specs=pl.BlockSpec((ROWS_B, D), lambda j: (j, 0)),
    out_shape=jax.ShapeDtypeStruct((N_PAD, D), _f32),
)


def _tc2_body(p_ref, d_ref, w2_ref, b2_ref, wd_ref, bd_ref, o_ref):
  a = p_ref[0] + p_ref[1]
  d = jnp.maximum(d_ref[0] + d_ref[1], 1.0)
  a = a / d
  h = jnp.maximum(
      jnp.dot(a, w2_ref[...], preferred_element_type=_f32) + b2_ref[...], 0.0)
  lg = jnp.dot(h, wd_ref[...], preferred_element_type=_f32) + bd_ref[...]
  m = jnp.max(lg, axis=-1, keepdims=True)
  e = jnp.exp(lg - m)
  o_ref[...] = e / jnp.sum(e, axis=-1, keepdims=True)


_tc2 = pl.pallas_call(
    _tc2_body,
    grid=(GRID,),
    in_specs=[
        pl.BlockSpec((NC, ROWS_B, D), lambda j: (0, j, 0)),
        pl.BlockSpec((NC, ROWS_B, 1), lambda j: (0, j, 0)),
        pl.BlockSpec((D, D), lambda j: (0, 0)),
        pl.BlockSpec((1, D), lambda j: (0, 0)),
        pl.BlockSpec((D, NUM_CLASSES), lambda j: (0, 0)),
        pl.BlockSpec((1, NUM_CLASSES), lambda j: (0, 0)),
    ],
    out_specs=pl.BlockSpec((ROWS_B, NUM_CLASSES), lambda j: (j, 0)),
    out_shape=jax.ShapeDtypeStruct((N_PAD, NUM_CLASSES), _f32),
)


def kernel(x, edge_index, W1, b1, W2, b2, Wd, bd):
  src = edge_index[0].reshape(NW, NCH, CHUNK)
  dst = edge_index[1].reshape(NW, NCH, CHUNK)
  agg1p, degp = _sc_agg_deg(x, src, dst)
  degp = degp.reshape(NC, N_PAD, 1)
  h1 = _tc1(agg1p, degp, W1, b1.reshape(1, D))
  agg2p, = _sc_agg(h1, src, dst)
  out = _tc2(agg2p, degp, W2, b2.reshape(1, D), Wd, bd.reshape(1, NUM_CLASSES))
  return out[:N_NODES]
